# baseline (speedup 1.0000x reference)
import functools

import jax
import jax.numpy as jnp
from jax import lax
from jax.experimental import pallas as pl
from jax.experimental.pallas import tpu as pltpu
from jax.experimental.pallas import tpu_sc as plsc

NC = 2
NS = 16
NW = NC * NS
LANES = 16
CHUNK = 2000
NBUF = 4


@functools.partial(jax.jit, static_argnames=("n_pad",))
def _agg2(h, edges_flat, wvec, n_pad):
  n = h.shape[0]
  e_tot = edges_flat.shape[0] // 2
  ew = e_tot // NW
  n_chunks = ew // CHUNK
  n_quads = n_chunks // NBUF
  s_sl = n_pad // NS
  h_sl = s_sl // 2

  def _pieces(total):
    out, off = [], 0
    while off < total:
      out.append((off, min(CHUNK, total - off)))
      off += out[-1][1]
    return out

  pieces = _pieces(s_sl)
  half_pieces = _pieces(h_sl)

  mesh = plsc.VectorSubcoreMesh(core_axis_name="c", subcore_axis_name="s")

  @functools.partial(
      pl.kernel,
      out_type=(
          jax.ShapeDtypeStruct((n_pad,), jnp.float32),
          jax.ShapeDtypeStruct((NC * n_pad,), jnp.float32),
      ),
      mesh=mesh,
      compiler_params=pltpu.CompilerParams(needs_layout_passes=False),
      scratch_types=[
          pltpu.VMEM((n_pad,), jnp.float32),
          pltpu.VMEM_SHARED((n_pad,), jnp.float32),
          pltpu.VMEM((64,), jnp.float32),
          [pltpu.VMEM((CHUNK,), jnp.int32) for _ in range(NBUF)],
          [pltpu.VMEM((CHUNK,), jnp.int32) for _ in range(NBUF)],
          [pltpu.VMEM((CHUNK,), jnp.float32) for _ in range(NBUF)],
          [pltpu.SemaphoreType.DMA for _ in range(NBUF)],
          [pltpu.SemaphoreType.DMA for _ in range(NBUF)],
          pltpu.SemaphoreType.DMA,
          pltpu.SemaphoreType.REGULAR,
      ],
  )
  def agg_kernel(h_hbm, edges_hbm, w_hbm, out_hbm, xch_hbm, h_loc, acc_sh,
                 w_loc, srcb, dstb, valb, ld, st, ldh, xsem):
    c = lax.axis_index("c")
    s = lax.axis_index("s")
    wid = s * NC + c
    base_n = s * s_sl
    base_e = wid * ew

    def _issue_ld(g, b):
      off = base_e + lax.rem(g, n_chunks) * CHUNK
      pltpu.async_copy(edges_hbm.at[pl.ds(off, CHUNK)], srcb[b], ld[b])
      pltpu.async_copy(edges_hbm.at[pl.ds(e_tot + off, CHUNK)], dstb[b], ld[b])

    def _wait_ld(b):
      pltpu.make_async_copy(edges_hbm.at[pl.ds(0, CHUNK)], srcb[b],
                            ld[b]).wait()
      pltpu.make_async_copy(edges_hbm.at[pl.ds(0, CHUNK)], dstb[b],
                            ld[b]).wait()

    def _issue_st(b):
      pltpu.async_copy(valb[b], acc_sh.at[dstb[b]], st[b], add=True)

    def _wait_st(b):
      pltpu.make_async_copy(valb[b], acc_sh.at[dstb[b]], st[b]).wait()

    def _gather(b):
      def body(i, carry):
        idx = srcb[b][pl.ds(i * LANES, LANES)]
        valb[b][pl.ds(i * LANES, LANES)] = plsc.load_gather(h_loc, [idx])
        return carry
      lax.fori_loop(0, CHUNK // LANES, body, 0)

    def _zero_acc_slice():
      def _zero(i, carry):
        valb[0][pl.ds(i * LANES, LANES)] = jnp.zeros((LANES,), jnp.float32)
        return carry
      lax.fori_loop(0, CHUNK // LANES, _zero, 0)
      for off, sz in pieces:
        pltpu.sync_copy(valb[0].at[pl.ds(0, sz)],
                        acc_sh.at[pl.ds(base_n + off, sz)])

    def _edge_pipeline(first):
      for g in range(NBUF):
        b = g % NBUF
        if first or g >= 2:
          _wait_ld(b)
        if g >= 2:
          _wait_st((g + 2) % NBUF)
        _issue_ld(g + 2, (g + 2) % NBUF)
        _gather(b)
        _issue_st(b)

      def _quad(q, carry):
        g0 = q * NBUF
        for b in range(NBUF):
          g = g0 + b
          _wait_ld(b)
          _wait_st((b + 2) % NBUF)
          _issue_ld(g + 2, (b + 2) % NBUF)
          _gather(b)
          _issue_st(b)
        return carry
      lax.fori_loop(1, n_quads, _quad, 0)

      _wait_ld(0)
      _wait_ld(1)
      _wait_st(2)
      _wait_st(3)

    def _stage_partials():
      for off, sz in pieces:
        pltpu.sync_copy(acc_sh.at[pl.ds(base_n + off, sz)],
                        valb[0].at[pl.ds(0, sz)])
        pltpu.sync_copy(valb[0].at[pl.ds(0, sz)],
                        xch_hbm.at[pl.ds(c * n_pad + base_n + off, sz)])
      pl.semaphore_signal(xsem, 1, core_index=1 - c)
      pl.semaphore_wait(xsem, 1)

    def _combine_half(w_self_idx):
      w_self_v = w_loc[pl.ds(2 * w_self_idx * LANES, LANES)]
      w_neigh_v = w_loc[pl.ds((2 * w_self_idx + 1) * LANES, LANES)]
      base_h = base_n + c * h_sl
      for off, sz in half_pieces:
        pltpu.sync_copy(acc_sh.at[pl.ds(base_h + off, sz)],
                        valb[0].at[pl.ds(0, sz)])
        pltpu.sync_copy(xch_hbm.at[pl.ds((1 - c) * n_pad + base_h + off, sz)],
                        valb[1].at[pl.ds(0, sz)])
        def body(i, carry):
          og = jnp.minimum(i * LANES, sz - LANES)
          a = valb[0][pl.ds(og, LANES)]
          b = valb[1][pl.ds(og, LANES)]
          hv = h_loc[pl.ds(base_h + off + og, LANES)]
          valb[2][pl.ds(og, LANES)] = (
              w_self_v * hv + w_neigh_v * (a + b))
          return carry
        lax.fori_loop(0, (sz + LANES - 1) // LANES, body, 0)
        pltpu.sync_copy(valb[2].at[pl.ds(0, sz)],
                        out_hbm.at[pl.ds(base_h + off, sz)])

    _issue_ld(0, 0)
    _issue_ld(1, 1)
    h_cp = pltpu.async_copy(h_hbm, h_loc.at[pl.ds(0, n)], ldh)
    pltpu.sync_copy(w_hbm, w_loc)
    _zero_acc_slice()
    plsc.subcore_barrier()
    h_cp.wait()

    _edge_pipeline(first=True)
    plsc.subcore_barrier()
    _stage_partials()

    _combine_half(0)
    _zero_acc_slice()
    plsc.subcore_barrier()
    pl.semaphore_signal(xsem, 1, core_index=1 - c)
    pl.semaphore_wait(xsem, 1)
    pltpu.sync_copy(out_hbm.at[pl.ds(0, n)], h_loc.at[pl.ds(0, n)])

    _edge_pipeline(first=False)
    plsc.subcore_barrier()
    _stage_partials()

    _combine_half(1)

  return agg_kernel(h, edges_flat, wvec)


def kernel(x, edge_index, W_self, W_neigh):
  n, d = x.shape
  num_hop = W_self.shape[0]
  assert d == 1 and num_hop == 2

  n_pad = -(-(n + 1) // (NS * 8)) * (NS * 8)

  e = edge_index.shape[1]
  e_pad = -(-e // (NW * NBUF * CHUNK)) * (NW * NBUF * CHUNK)
  if e_pad != e:
    pad = jnp.zeros((2, e_pad - e), jnp.int32).at[1, :].set(n)
    edge_index = jnp.concatenate([edge_index, pad], axis=1)
  edges_flat = edge_index.reshape(-1)

  wvec = jnp.repeat(
      jnp.stack([W_self[0, 0, 0], W_neigh[0, 0, 0],
                 W_self[1, 0, 0], W_neigh[1, 0, 0]]), LANES)

  h2, _ = _agg2(x[:, 0], edges_flat, wvec, n_pad)
  return h2[:n, None]


# speedup vs baseline: 1.0039x; 1.0039x over previous
import functools

import jax
import jax.numpy as jnp
from jax import lax
from jax.experimental import pallas as pl
from jax.experimental.pallas import tpu as pltpu
from jax.experimental.pallas import tpu_sc as plsc

NC = 2
NS = 16
NW = NC * NS
LANES = 16
CHUNK = 2000
NBUF = 4


@functools.partial(jax.jit, static_argnames=("n_pad",))
def _hop(h, edges_flat, n_pad):
  n = h.shape[0]
  e_tot = edges_flat.shape[0] // 2
  ew = e_tot // NW
  n_chunks = ew // CHUNK
  n_quads = n_chunks // NBUF
  s_sl = n_pad // NS

  mesh = plsc.VectorSubcoreMesh(core_axis_name="c", subcore_axis_name="s")

  @functools.partial(
      pl.kernel,
      out_type=jax.ShapeDtypeStruct((NC * n_pad,), jnp.float32),
      mesh=mesh,
      compiler_params=pltpu.CompilerParams(needs_layout_passes=False),
      scratch_types=[
          pltpu.VMEM((n,), jnp.float32),
          pltpu.VMEM_SHARED((n_pad,), jnp.float32),
          [pltpu.VMEM((CHUNK,), jnp.int32) for _ in range(NBUF)],
          [pltpu.VMEM((CHUNK,), jnp.int32) for _ in range(NBUF)],
          [pltpu.VMEM((CHUNK,), jnp.float32) for _ in range(NBUF)],
          [pltpu.SemaphoreType.DMA for _ in range(NBUF)],
          [pltpu.SemaphoreType.DMA for _ in range(NBUF)],
          pltpu.SemaphoreType.DMA,
      ],
  )
  def hop_kernel(h_hbm, edges_hbm, out_hbm, h_loc, acc_sh,
                 srcb, dstb, valb, ld, st, ldh):
    c = lax.axis_index("c")
    s = lax.axis_index("s")
    wid = s * NC + c
    base_n = s * s_sl
    base_e = wid * ew

    def _issue_ld(g, b):
      off = base_e + lax.rem(g, n_chunks) * CHUNK
      pltpu.async_copy(edges_hbm.at[pl.ds(off, CHUNK)], srcb[b], ld[b])
      pltpu.async_copy(edges_hbm.at[pl.ds(e_tot + off, CHUNK)], dstb[b], ld[b])

    def _wait_ld(b):
      pltpu.make_async_copy(edges_hbm.at[pl.ds(0, CHUNK)], srcb[b],
                            ld[b]).wait()
      pltpu.make_async_copy(edges_hbm.at[pl.ds(0, CHUNK)], dstb[b],
                            ld[b]).wait()

    def _issue_st(b):
      pltpu.async_copy(valb[b], acc_sh.at[dstb[b]], st[b], add=True)

    def _wait_st(b):
      pltpu.make_async_copy(valb[b], acc_sh.at[dstb[b]], st[b]).wait()

    def _gather(b):
      def body(i, carry):
        idx = srcb[b][pl.ds(i * LANES, LANES)]
        valb[b][pl.ds(i * LANES, LANES)] = plsc.load_gather(h_loc, [idx])
        return carry
      lax.fori_loop(0, CHUNK // LANES, body, 0)

    _issue_ld(0, 0)
    h_cp = pltpu.async_copy(h_hbm, h_loc, ldh)

    def _zero(i, carry):
      valb[0][pl.ds(i * LANES, LANES)] = jnp.zeros((LANES,), jnp.float32)
      return carry
    lax.fori_loop(0, CHUNK // LANES, _zero, 0)
    off = 0
    while off < s_sl:
      piece = min(CHUNK, s_sl - off)
      pltpu.sync_copy(valb[0].at[pl.ds(0, piece)],
                      acc_sh.at[pl.ds(base_n + off, piece)])
      off += piece
    plsc.subcore_barrier()
    h_cp.wait()

    for g in range(NBUF):
      b = g % NBUF
      _wait_ld(b)
      if g >= 3:
        _wait_st((g + 1) % NBUF)
      _issue_ld(g + 1, (g + 1) % NBUF)
      _gather(b)
      _issue_st(b)

    def _quad(q, carry):
      g0 = q * NBUF
      for b in range(NBUF):
        g = g0 + b
        _wait_ld(b)
        _wait_st((b + 1) % NBUF)
        _issue_ld(g + 1, (b + 1) % NBUF)
        _gather(b)
        _issue_st(b)
      return carry
    lax.fori_loop(1, n_quads, _quad, 0)

    _wait_ld(0)
    _wait_st(1)
    _wait_st(2)
    _wait_st(3)
    plsc.subcore_barrier()

    off = 0
    while off < s_sl:
      piece = min(CHUNK, s_sl - off)
      pltpu.sync_copy(acc_sh.at[pl.ds(base_n + off, piece)],
                      valb[0].at[pl.ds(0, piece)])
      pltpu.sync_copy(valb[0].at[pl.ds(0, piece)],
                      out_hbm.at[pl.ds(c * n_pad + base_n + off, piece)])
      off += piece

  return hop_kernel(h, edges_flat)


def kernel(x, edge_index, W_self, W_neigh):
  n, d = x.shape
  num_hop = W_self.shape[0]
  assert d == 1

  n_pad = -(-(n + 1) // (NS * 8)) * (NS * 8)

  e = edge_index.shape[1]
  e_pad = -(-e // (NW * NBUF * CHUNK)) * (NW * NBUF * CHUNK)
  if e_pad != e:
    pad = jnp.zeros((2, e_pad - e), jnp.int32).at[1, :].set(n)
    edge_index = jnp.concatenate([edge_index, pad], axis=1)
  edges_flat = edge_index.reshape(-1)

  h = x[:, 0]
  for i in range(num_hop):
    parts = _hop(h, edges_flat, n_pad)
    acc = parts[:n] + parts[n_pad:n_pad + n]
    h = W_self[i, 0, 0] * h + W_neigh[i, 0, 0] * acc
  return h[:, None]


# speedup vs baseline: 1.0123x; 1.0084x over previous
import functools

import jax
import jax.numpy as jnp
from jax import lax
from jax.experimental import pallas as pl
from jax.experimental.pallas import tpu as pltpu
from jax.experimental.pallas import tpu_sc as plsc

NC = 2
NS = 16
NW = NC * NS
LANES = 16
CHUNK = 2000
NBUF = 4


@functools.partial(jax.jit, static_argnames=("n_pad",))
def _hop(h, edges_flat, n_pad):
  n = h.shape[0]
  e_tot = edges_flat.shape[0] // 2
  ew = e_tot // NW
  n_chunks = ew // CHUNK
  n_quads = n_chunks // NBUF
  s_sl = n_pad // NS

  mesh = plsc.VectorSubcoreMesh(core_axis_name="c", subcore_axis_name="s")

  @functools.partial(
      pl.kernel,
      out_type=jax.ShapeDtypeStruct((NC * n_pad,), jnp.float32),
      mesh=mesh,
      compiler_params=pltpu.CompilerParams(needs_layout_passes=False),
      scratch_types=[
          pltpu.VMEM((n,), jnp.float32),
          pltpu.VMEM_SHARED((n_pad,), jnp.float32),
          [pltpu.VMEM((CHUNK,), jnp.int32) for _ in range(NBUF)],
          [pltpu.VMEM((CHUNK,), jnp.int32) for _ in range(NBUF)],
          [pltpu.VMEM((CHUNK,), jnp.float32) for _ in range(NBUF)],
          [pltpu.SemaphoreType.DMA for _ in range(NBUF)],
          [pltpu.SemaphoreType.DMA for _ in range(NBUF)],
          pltpu.SemaphoreType.DMA,
      ],
  )
  def hop_kernel(h_hbm, edges_hbm, out_hbm, h_loc, acc_sh,
                 srcb, dstb, valb, ld, st, ldh):
    c = lax.axis_index("c")
    s = lax.axis_index("s")
    wid = s * NC + c
    base_n = s * s_sl
    base_e = wid * ew

    def _issue_ld(g, b):
      off = base_e + lax.rem(g, n_chunks) * CHUNK
      pltpu.async_copy(edges_hbm.at[pl.ds(off, CHUNK)], srcb[b], ld[b])
      pltpu.async_copy(edges_hbm.at[pl.ds(e_tot + off, CHUNK)], dstb[b], ld[b])

    def _wait_ld(b):
      pltpu.make_async_copy(edges_hbm.at[pl.ds(0, CHUNK)], srcb[b],
                            ld[b]).wait()
      pltpu.make_async_copy(edges_hbm.at[pl.ds(0, CHUNK)], dstb[b],
                            ld[b]).wait()

    def _issue_st(b):
      pltpu.async_copy(valb[b], acc_sh.at[dstb[b]], st[b], add=True)

    def _wait_st(b):
      pltpu.make_async_copy(valb[b], acc_sh.at[dstb[b]], st[b]).wait()

    def _gather(b):
      def body(i, carry):
        idx = srcb[b][pl.ds(i * LANES, LANES)]
        valb[b][pl.ds(i * LANES, LANES)] = plsc.load_gather(h_loc, [idx])
        return carry
      lax.fori_loop(0, CHUNK // LANES, body, 0)

    _issue_ld(0, 0)
    _issue_ld(1, 1)
    h_cp = pltpu.async_copy(h_hbm, h_loc, ldh)

    def _zero(i, carry):
      valb[0][pl.ds(i * LANES, LANES)] = jnp.zeros((LANES,), jnp.float32)
      return carry
    lax.fori_loop(0, CHUNK // LANES, _zero, 0)
    off = 0
    while off < s_sl:
      piece = min(CHUNK, s_sl - off)
      pltpu.sync_copy(valb[0].at[pl.ds(0, piece)],
                      acc_sh.at[pl.ds(base_n + off, piece)])
      off += piece
    plsc.subcore_barrier()
    h_cp.wait()

    for g in range(NBUF):
      b = g % NBUF
      _wait_ld(b)
      if g >= 2:
        _wait_st((g + 2) % NBUF)
      _issue_ld(g + 2, (g + 2) % NBUF)
      _gather(b)
      _issue_st(b)

    def _quad(q, carry):
      g0 = q * NBUF
      for b in range(NBUF):
        g = g0 + b
        _wait_ld(b)
        _wait_st((b + 2) % NBUF)
        _issue_ld(g + 2, (b + 2) % NBUF)
        _gather(b)
        _issue_st(b)
      return carry
    lax.fori_loop(1, n_quads, _quad, 0)

    _wait_ld(0)
    _wait_ld(1)
    _wait_st(2)
    _wait_st(3)
    plsc.subcore_barrier()

    off = 0
    while off < s_sl:
      piece = min(CHUNK, s_sl - off)
      pltpu.sync_copy(acc_sh.at[pl.ds(base_n + off, piece)],
                      valb[0].at[pl.ds(0, piece)])
      pltpu.sync_copy(valb[0].at[pl.ds(0, piece)],
                      out_hbm.at[pl.ds(c * n_pad + base_n + off, piece)])
      off += piece

  return hop_kernel(h, edges_flat)


def kernel(x, edge_index, W_self, W_neigh):
  n, d = x.shape
  num_hop = W_self.shape[0]
  assert d == 1

  n_pad = -(-(n + 1) // (NS * 8)) * (NS * 8)

  e = edge_index.shape[1]
  e_pad = -(-e // (NW * NBUF * CHUNK)) * (NW * NBUF * CHUNK)
  if e_pad != e:
    pad = jnp.zeros((2, e_pad - e), jnp.int32).at[1, :].set(n)
    edge_index = jnp.concatenate([edge_index, pad], axis=1)
  edges_flat = edge_index.reshape(-1)

  h = x[:, 0]
  for i in range(num_hop):
    parts = _hop(h, edges_flat, n_pad)
    acc = parts[:n] + parts[n_pad:n_pad + n]
    h = W_self[i, 0, 0] * h + W_neigh[i, 0, 0] * acc
  return h[:, None]
